# baseline (speedup 1.0000x reference)
import jax
import jax.numpy as jnp
from jax import lax
from jax.experimental import pallas as pl
from jax.experimental.pallas import tpu as pltpu
from jax.experimental.pallas import tpu_sc as plsc

_B, _D, _H, _W, _M = 8, 10, 188, 188, 500
_HW = _H * _W
_MP = 512
_NT = 16
_CHUNK = 128
_NCHUNKS = _D * _B * (_MP // _CHUNK)
_CPT = _NCHUNKS // _NT
_NV = _CHUNK // 16


def _sc_loss_body(outflat, indflat, maskflat, tgtflat, part, out,
                  ind_v, mask_v, idxp_v, idxt_v, pred_v, tgt_v,
                  red_v, sum_v, psem, tsem):
    core = lax.axis_index("c")
    sub = lax.axis_index("s")

    @pl.when(core == 0)
    def _body():
        pltpu.sync_copy(indflat, ind_v)
        pltpu.sync_copy(maskflat, mask_v)

        lanes = lax.iota(jnp.int32, 16)

        handles = []
        for k in range(_CPT):
            c = sub * _CPT + k
            d = c // 32
            r = c % 32
            b = r // 4
            mc = r % 4
            pbase = (b * _D + d) * _HW
            ioff = b * _MP + mc * _CHUNK
            for j in range(_NV):
                iv = ind_v[pl.ds(ioff + j * 16, 16)]
                idxp_v[k, pl.ds(j * 16, 16)] = iv + pbase
                mvec = mc * _CHUNK + j * 16 + lanes
                mclamp = jnp.minimum(mvec, _M - 1)
                idxt_v[k, pl.ds(j * 16, 16)] = (b * _M + mclamp) * _D + d
            hp = pltpu.async_copy(outflat.at[idxp_v.at[k]], pred_v.at[k], psem)
            ht = pltpu.async_copy(tgtflat.at[idxt_v.at[k]], tgt_v.at[k], tsem)
            handles.append((hp, ht))

        for hp, ht in handles:
            hp.wait()
            ht.wait()

        acc = jnp.zeros((16,), jnp.float32)
        msum = jnp.float32(0.0)
        for k in range(_CPT):
            c = sub * _CPT + k
            d = c // 32
            r = c % 32
            b = r // 4
            mc = r % 4
            ioff = b * _MP + mc * _CHUNK
            csum = jnp.zeros((16,), jnp.float32)
            mk = jnp.zeros((16,), jnp.float32)
            for j in range(_NV):
                p = pred_v[k, pl.ds(j * 16, 16)]
                t = tgt_v[k, pl.ds(j * 16, 16)]
                w = mask_v[pl.ds(ioff + j * 16, 16)]
                wm = jnp.where(t == t, w, jnp.float32(0.0))
                csum = csum + jnp.abs(p * wm - t * wm)
                mk = mk + w
            sval = jnp.sum(csum)
            msum = msum + jnp.where(d == 0, jnp.sum(mk), jnp.float32(0.0))
            acc = acc + jnp.where(lanes == d, sval, jnp.float32(0.0))

        acc = acc + jnp.where(lanes == _D, msum, jnp.float32(0.0))

        red_v[...] = acc
        pltpu.sync_copy(red_v, part.at[sub])
        plsc.subcore_barrier()

        @pl.when(sub == 0)
        def _final():
            pltpu.sync_copy(part, sum_v)
            tot = jnp.zeros((16,), jnp.float32)
            for i in range(_NT):
                tot = tot + sum_v[i, :]
            num_v = jnp.full((16,), tot[_D], jnp.float32)
            denom = jnp.maximum(num_v, jnp.float32(1.0))
            red_v[...] = tot / denom
            pltpu.sync_copy(red_v, out)


def kernel(output, mask, ind, target):
    outflat = output.reshape(-1)
    indpad = jnp.pad(ind.astype(jnp.int32), ((0, 0), (0, _MP - _M)))
    maskpad = jnp.pad(mask.astype(jnp.float32), ((0, 0), (0, _MP - _M)))
    tgtflat = target.reshape(-1)

    mesh = plsc.VectorSubcoreMesh(core_axis_name="c", subcore_axis_name="s")
    f = pl.kernel(
        _sc_loss_body,
        out_type=(jax.ShapeDtypeStruct((_NT, 16), jnp.float32),
                  jax.ShapeDtypeStruct((16,), jnp.float32)),
        mesh=mesh,
        compiler_params=pltpu.CompilerParams(needs_layout_passes=False),
        scratch_types=[
            pltpu.VMEM((_B * _MP,), jnp.int32),
            pltpu.VMEM((_B * _MP,), jnp.float32),
            pltpu.VMEM((_CPT, _CHUNK), jnp.int32),
            pltpu.VMEM((_CPT, _CHUNK), jnp.int32),
            pltpu.VMEM((_CPT, _CHUNK), jnp.float32),
            pltpu.VMEM((_CPT, _CHUNK), jnp.float32),
            pltpu.VMEM((16,), jnp.float32),
            pltpu.VMEM((_NT, 16), jnp.float32),
            pltpu.SemaphoreType.DMA,
            pltpu.SemaphoreType.DMA,
        ],
    )
    _, res = f(outflat, indpad.reshape(-1), maskpad.reshape(-1), tgtflat)
    return res[:_D]


# speedup vs baseline: 1.3038x; 1.3038x over previous
import jax
import jax.numpy as jnp
from jax import lax
from jax.experimental import pallas as pl
from jax.experimental.pallas import tpu as pltpu
from jax.experimental.pallas import tpu_sc as plsc

_B, _D, _H, _W, _M = 8, 10, 188, 188, 500
_HW = _H * _W
_MP = 512
_NT = 16
_CHUNK = 128
_NCHUNKS = _D * _B * (_MP // _CHUNK)
_CPT = _NCHUNKS // _NT
_NV = _CHUNK // 16


def _sc_loss_body(outflat, indflat, maskflat, tgtflat, part, out,
                  ind_v, mask_v, idxp_v, idxt_v, pred_v, tgt_v,
                  red_v, sum_v, psem, tsem):
    core = lax.axis_index("c")
    sub = lax.axis_index("s")

    @pl.when(core == 0)
    def _body():
        pltpu.sync_copy(indflat, ind_v)
        pltpu.sync_copy(maskflat, mask_v)

        lanes = lax.iota(jnp.int32, 16)

        handles = []
        for k in range(_CPT):
            c = sub * _CPT + k
            d = c // 32
            r = c % 32
            b = r // 4
            mc = r % 4
            ioff = b * _MP + mc * _CHUNK
            pbase = d * (_H * _B * _W)
            for j in range(_NV):
                iv = ind_v[pl.ds(ioff + j * 16, 16)]
                idxp_v[k, pl.ds(j * 16, 16)] = iv + pbase
                mvec = mc * _CHUNK + j * 16 + lanes
                mclamp = jnp.minimum(mvec, _M - 1)
                idxt_v[k, pl.ds(j * 16, 16)] = (b * _M + mclamp) * _D + d
            hp = pltpu.async_copy(outflat.at[idxp_v.at[k]], pred_v.at[k], psem)
            ht = pltpu.async_copy(tgtflat.at[idxt_v.at[k]], tgt_v.at[k], tsem)
            handles.append((hp, ht))

        for hp, ht in handles:
            hp.wait()
            ht.wait()

        acc = jnp.zeros((16,), jnp.float32)
        msum = jnp.float32(0.0)
        for k in range(_CPT):
            c = sub * _CPT + k
            d = c // 32
            r = c % 32
            b = r // 4
            mc = r % 4
            ioff = b * _MP + mc * _CHUNK
            csum = jnp.zeros((16,), jnp.float32)
            mk = jnp.zeros((16,), jnp.float32)
            for j in range(_NV):
                p = pred_v[k, pl.ds(j * 16, 16)]
                t = tgt_v[k, pl.ds(j * 16, 16)]
                w = mask_v[pl.ds(ioff + j * 16, 16)]
                wm = jnp.where(t == t, w, jnp.float32(0.0))
                csum = csum + jnp.abs(p * wm - t * wm)
                mk = mk + w
            sval = jnp.sum(csum)
            msum = msum + jnp.where(d == 0, jnp.sum(mk), jnp.float32(0.0))
            acc = acc + jnp.where(lanes == d, sval, jnp.float32(0.0))

        acc = acc + jnp.where(lanes == _D, msum, jnp.float32(0.0))

        red_v[...] = acc
        pltpu.sync_copy(red_v, part.at[sub])
        plsc.subcore_barrier()

        @pl.when(sub == 0)
        def _final():
            pltpu.sync_copy(part, sum_v)
            tot = jnp.zeros((16,), jnp.float32)
            for i in range(_NT):
                tot = tot + sum_v[i, :]
            num_v = jnp.full((16,), tot[_D], jnp.float32)
            denom = jnp.maximum(num_v, jnp.float32(1.0))
            red_v[...] = tot / denom
            pltpu.sync_copy(red_v, out)


def kernel(output, mask, ind, target):
    outflat = jnp.transpose(output, (1, 2, 0, 3)).reshape(-1)
    ind32 = ind.astype(jnp.int32)
    h = ind32 // _W
    w = ind32 - h * _W
    base = h * (_B * _W) + jnp.arange(_B, dtype=jnp.int32)[:, None] * _W + w
    indpad = jnp.pad(base, ((0, 0), (0, _MP - _M)))
    maskpad = jnp.pad(mask.astype(jnp.float32), ((0, 0), (0, _MP - _M)))
    tgtflat = target.reshape(-1)

    mesh = plsc.VectorSubcoreMesh(core_axis_name="c", subcore_axis_name="s")
    f = pl.kernel(
        _sc_loss_body,
        out_type=(jax.ShapeDtypeStruct((_NT, 16), jnp.float32),
                  jax.ShapeDtypeStruct((16,), jnp.float32)),
        mesh=mesh,
        compiler_params=pltpu.CompilerParams(needs_layout_passes=False),
        scratch_types=[
            pltpu.VMEM((_B * _MP,), jnp.int32),
            pltpu.VMEM((_B * _MP,), jnp.float32),
            pltpu.VMEM((_CPT, _CHUNK), jnp.int32),
            pltpu.VMEM((_CPT, _CHUNK), jnp.int32),
            pltpu.VMEM((_CPT, _CHUNK), jnp.float32),
            pltpu.VMEM((_CPT, _CHUNK), jnp.float32),
            pltpu.VMEM((16,), jnp.float32),
            pltpu.VMEM((_NT, 16), jnp.float32),
            pltpu.SemaphoreType.DMA,
            pltpu.SemaphoreType.DMA,
        ],
    )
    _, res = f(outflat, indpad.reshape(-1), maskpad.reshape(-1), tgtflat)
    return res[:_D]
